# baseline (speedup 1.0000x reference)
import functools

import jax
import jax.numpy as jnp
from jax import lax
from jax.experimental import pallas as pl

_F32 = jnp.float32
_DN_T = (((1,), (1,)), ((), ()))
_DN = (((1,), (0,)), ((), ()))


def _dot(x, w):
    return lax.dot_general(x, w, _DN_T, preferred_element_type=_F32)


def _fps_body(npoint, xyz_ref, out_ref):
    xyz = xyz_ref[...]
    b, _, n = xyz.shape
    x = xyz[:, 0, :]
    y = xyz[:, 1, :]
    z = xyz[:, 2, :]
    lane = lax.broadcasted_iota(jnp.int32, (b, n), 1)

    def step(i, state):
        distance, farthest = state
        sel = lane == farthest
        cx = jnp.sum(jnp.where(sel, x, 0.0), axis=1, keepdims=True)
        cy = jnp.sum(jnp.where(sel, y, 0.0), axis=1, keepdims=True)
        cz = jnp.sum(jnp.where(sel, z, 0.0), axis=1, keepdims=True)
        out_ref[pl.ds(i, 1)] = jnp.concatenate([cx, cy, cz], axis=1)[None]
        dx = x - cx
        dy = y - cy
        dz = z - cz
        dist = (dx * dx + dy * dy) + dz * dz
        distance = jnp.minimum(distance, dist)
        farthest = jnp.argmax(distance, axis=1).astype(jnp.int32)[:, None]
        return distance, farthest

    init = (jnp.full((b, n), 1e10, _F32), jnp.zeros((b, 1), jnp.int32))
    lax.fori_loop(0, npoint, step, init)


def _fps(xyz_t, npoint):
    b, _, n = xyz_t.shape
    out = pl.pallas_call(
        functools.partial(_fps_body, npoint),
        out_shape=jax.ShapeDtypeStruct((npoint, b, 3), _F32),
    )(xyz_t)
    return jnp.transpose(out, (1, 0, 2))


def _sa_body(r2, sb, ns, nlayers, *refs):
    pts_ref, xyzr_ref, feat_ref, ctr_ref = refs[:4]
    wrefs = refs[4:4 + 2 * nlayers]
    out_ref = refs[-1]
    pts = pts_ref[0]
    ctr = ctr_ref[0]
    n = pts.shape[1]
    d2 = None
    for c in range(3):
        diff = ctr[:, c:c + 1] - pts[c:c + 1, :]
        t = diff * diff
        d2 = t if d2 is None else d2 + t
    mask = d2 <= r2
    cnt = mask.astype(jnp.int32)
    s = 1
    while s < n:
        cnt = cnt + jnp.concatenate(
            [jnp.zeros((sb, s), jnp.int32), cnt[:, :n - s]], axis=1)
        s *= 2
    rank = jnp.where(mask, cnt - 1, -1)
    counts = cnt[:, n - 1:n]
    kio = lax.broadcasted_iota(jnp.int32, (sb, ns, n), 1)
    oh = (rank[:, None, :] == kio).astype(_F32)
    vals = jnp.concatenate([xyzr_ref[0], feat_ref[0]], axis=1)
    cdim = vals.shape[1]
    g = lax.dot_general(oh.reshape(sb * ns, n), vals, _DN,
                        preferred_element_type=_F32)
    g3 = g.reshape(sb, ns, cdim)
    valid = lax.broadcasted_iota(jnp.int32, (sb, ns, 1), 1) < counts[:, :, None]
    g3 = jnp.where(valid, g3, g3[:, 0:1, :])
    cpad = jnp.concatenate([ctr, jnp.zeros((sb, cdim - 3), _F32)], axis=1)
    h = (g3 - cpad[:, None, :]).reshape(sb * ns, cdim)
    for li in range(nlayers):
        h = jnp.maximum(_dot(h, wrefs[2 * li][...]) + wrefs[2 * li + 1][...], 0.0)
    out_ref[0] = jnp.max(h.reshape(sb, ns, h.shape[1]), axis=1)


def _sa(pts_t, xyz_r, feat_r, ctr, ws, bs, radius, sb):
    b, _, n = pts_t.shape
    s_tot = ctr.shape[1]
    cin = feat_r.shape[2]
    o_last = ws[-1].shape[0]
    in_specs = [
        pl.BlockSpec((1, 3, n), lambda i, j: (i, 0, 0)),
        pl.BlockSpec((1, n, 3), lambda i, j: (i, 0, 0)),
        pl.BlockSpec((1, n, cin), lambda i, j: (i, 0, 0)),
        pl.BlockSpec((1, sb, 3), lambda i, j: (i, j, 0)),
    ]
    args = [pts_t, xyz_r, feat_r, ctr]
    for w, bias in zip(ws, bs):
        args += [w, bias.reshape(1, -1)]
        in_specs += [pl.BlockSpec(w.shape, lambda i, j: (0, 0)),
                     pl.BlockSpec((1, bias.shape[0]), lambda i, j: (0, 0))]
    return pl.pallas_call(
        functools.partial(_sa_body, float(radius) ** 2, sb, 64, len(ws)),
        grid=(b, s_tot // sb),
        in_specs=in_specs,
        out_specs=pl.BlockSpec((1, sb, o_last), lambda i, j: (i, j, 0)),
        out_shape=jax.ShapeDtypeStruct((b, s_tot, o_last), _F32),
    )(*args)


def _gsa_fp1_body(p2_ref, f2_ref, w1p_ref, w1f_ref, b1_ref, w2_ref, b2_ref,
                  w3_ref, b3_ref, a1_ref, ai_ref, c1_ref, a2_ref, c2_ref,
                  out_ref):
    f2 = f2_ref[0]
    h = jnp.maximum(_dot(p2_ref[0], w1p_ref[...])
                    + _dot(f2, w1f_ref[...]) + b1_ref[...], 0.0)
    h = jnp.maximum(_dot(h, w2_ref[...]) + b2_ref[...], 0.0)
    h = jnp.maximum(_dot(h, w3_ref[...]) + b3_ref[...], 0.0)
    f3 = jnp.max(h, axis=0, keepdims=True)
    g = jnp.maximum(_dot(f2, a1_ref[...])
                    + _dot(f3, ai_ref[...]) + c1_ref[...], 0.0)
    g = jnp.maximum(_dot(g, a2_ref[...]) + c2_ref[...], 0.0)
    out_ref[0] = g


def _gsa_fp1(p2_r, f2, gsa_w, gsa_b, fp1_w, fp1_b):
    b = p2_r.shape[0]
    args = [p2_r, f2,
            gsa_w[0][:, :3], gsa_w[0][:, 3:], gsa_b[0].reshape(1, -1),
            gsa_w[1], gsa_b[1].reshape(1, -1),
            gsa_w[2], gsa_b[2].reshape(1, -1),
            fp1_w[0][:, :256], fp1_w[0][:, 256:], fp1_b[0].reshape(1, -1),
            fp1_w[1], fp1_b[1].reshape(1, -1)]
    in_specs = [pl.BlockSpec((1, 128, 3), lambda i: (i, 0, 0)),
                pl.BlockSpec((1, 128, 256), lambda i: (i, 0, 0))]
    in_specs += [pl.BlockSpec(a.shape, lambda i: (0, 0)) for a in args[2:]]
    return pl.pallas_call(
        _gsa_fp1_body,
        grid=(b,),
        in_specs=in_specs,
        out_specs=pl.BlockSpec((1, 128, 256), lambda i: (i, 0, 0)),
        out_shape=jax.ShapeDtypeStruct((b, 128, 256), _F32),
    )(*args)


def _interp3(tgt_r, src_t, src_feat):
    d = None
    for c in range(3):
        diff = tgt_r[:, c:c + 1] - src_t[c:c + 1, :]
        t = diff * diff
        d = t if d is None else d + t
    lio = lax.broadcasted_iota(jnp.int32, d.shape, 1)
    inf = jnp.float32(jnp.inf)
    i1 = jnp.argmin(d, axis=1).astype(jnp.int32)[:, None]
    m1 = jnp.min(d, axis=1)[:, None]
    d2 = jnp.where(lio == i1, inf, d)
    i2 = jnp.argmin(d2, axis=1).astype(jnp.int32)[:, None]
    m2 = jnp.min(d2, axis=1)[:, None]
    d3 = jnp.where(lio == i2, inf, d2)
    i3 = jnp.argmin(d3, axis=1).astype(jnp.int32)[:, None]
    m3 = jnp.min(d3, axis=1)[:, None]
    wa = 1.0 / (m1 + 1e-8)
    wb = 1.0 / (m2 + 1e-8)
    wc = 1.0 / (m3 + 1e-8)
    tot = (wa + wb) + wc
    a = (jnp.where(lio == i1, wa / tot, 0.0)
         + jnp.where(lio == i2, wb / tot, 0.0)
         + jnp.where(lio == i3, wc / tot, 0.0))
    return lax.dot_general(a, src_feat, _DN, preferred_element_type=_F32)


def _fp2_body(p1_ref, p2t_ref, f1_ref, f2p_ref, a1_ref, ai_ref, b1_ref,
              w2_ref, b2_ref, out_ref):
    interp = _interp3(p1_ref[0], p2t_ref[0], f2p_ref[0])
    h = jnp.maximum(_dot(f1_ref[0], a1_ref[...])
                    + _dot(interp, ai_ref[...]) + b1_ref[...], 0.0)
    h = jnp.maximum(_dot(h, w2_ref[...]) + b2_ref[...], 0.0)
    out_ref[0] = h


def _fp2(p1_r, p2_t, f1, f2p, fp2_w, fp2_b):
    b = p1_r.shape[0]
    args = [p1_r, p2_t, f1, f2p,
            fp2_w[0][:, :128], fp2_w[0][:, 128:], fp2_b[0].reshape(1, -1),
            fp2_w[1], fp2_b[1].reshape(1, -1)]
    in_specs = [pl.BlockSpec((1, 512, 3), lambda i: (i, 0, 0)),
                pl.BlockSpec((1, 3, 128), lambda i: (i, 0, 0)),
                pl.BlockSpec((1, 512, 128), lambda i: (i, 0, 0)),
                pl.BlockSpec((1, 128, 256), lambda i: (i, 0, 0))]
    in_specs += [pl.BlockSpec(a.shape, lambda i: (0, 0)) for a in args[4:]]
    return pl.pallas_call(
        _fp2_body,
        grid=(b,),
        in_specs=in_specs,
        out_specs=pl.BlockSpec((1, 512, 128), lambda i: (i, 0, 0)),
        out_shape=jax.ShapeDtypeStruct((b, 512, 128), _F32),
    )(*args)


def _fp3_body(ptsr_ref, featr_ref, oh_ref, p1t_ref, f1p_ref,
              w1o_ref, w1f_ref, w1p_ref, w1i_ref, b1_ref,
              w2_ref, b2_ref, w3_ref, b3_ref,
              c1_ref, cb1_ref, c2_ref, cb2_ref, out_ref):
    ptsr = ptsr_ref[0]
    interp = _interp3(ptsr, p1t_ref[0], f1p_ref[0])
    h = jnp.maximum(_dot(oh_ref[0], w1o_ref[...])
                    + _dot(featr_ref[0], w1f_ref[...])
                    + _dot(ptsr, w1p_ref[...])
                    + _dot(interp, w1i_ref[...]) + b1_ref[...], 0.0)
    h = jnp.maximum(_dot(h, w2_ref[...]) + b2_ref[...], 0.0)
    h = jnp.maximum(_dot(h, w3_ref[...]) + b3_ref[...], 0.0)
    h = jnp.maximum(_dot(h, c1_ref[...]) + cb1_ref[...], 0.0)
    out = lax.dot_general(c2_ref[...], h, _DN_T,
                          preferred_element_type=_F32)
    out_ref[0] = out + cb2_ref[...]


def _fp3_cls(pts_r, feat_r, oh, p1_t, f1p, fp3_w, fp3_b, cls_w, cls_b):
    b, n, _ = pts_r.shape
    w1 = fp3_w[0]
    args = [pts_r, feat_r, oh, p1_t, f1p,
            w1[:, :16], w1[:, 16:19], w1[:, 19:22], w1[:, 22:],
            fp3_b[0].reshape(1, -1),
            fp3_w[1], fp3_b[1].reshape(1, -1),
            fp3_w[2], fp3_b[2].reshape(1, -1),
            cls_w[0], cls_b[0].reshape(1, -1),
            cls_w[1], cls_b[1].reshape(-1, 1)]
    in_specs = [pl.BlockSpec((1, n, 3), lambda i: (i, 0, 0)),
                pl.BlockSpec((1, n, 3), lambda i: (i, 0, 0)),
                pl.BlockSpec((1, 1, 16), lambda i: (i, 0, 0)),
                pl.BlockSpec((1, 3, 512), lambda i: (i, 0, 0)),
                pl.BlockSpec((1, 512, 128), lambda i: (i, 0, 0))]
    in_specs += [pl.BlockSpec(a.shape, lambda i: (0, 0)) for a in args[5:]]
    return pl.pallas_call(
        _fp3_body,
        grid=(b,),
        in_specs=in_specs,
        out_specs=pl.BlockSpec((1, 50, n), lambda i: (i, 0, 0)),
        out_shape=jax.ShapeDtypeStruct((b, 50, n), _F32),
    )(*args)


def kernel(points, features, class_ids, params):
    p = params
    pts_r = jnp.transpose(points, (0, 2, 1))
    feat_r = jnp.transpose(features, (0, 2, 1))
    oh = jax.nn.one_hot(class_ids, 16, dtype=_F32)[:, None, :]

    ctr1 = _fps(points, 512)
    f1 = _sa(points, pts_r, feat_r, ctr1, p['sa1_w'], p['sa1_b'], 0.2, 64)
    p1_t = jnp.transpose(ctr1, (0, 2, 1))
    ctr2 = _fps(p1_t, 128)
    f2 = _sa(p1_t, ctr1, f1, ctr2, p['sa2_w'], p['sa2_b'], 0.4, 32)
    f2p = _gsa_fp1(ctr2, f2, p['gsa_w'], p['gsa_b'], p['fp1_w'], p['fp1_b'])
    p2_t = jnp.transpose(ctr2, (0, 2, 1))
    f1p = _fp2(ctr1, p2_t, f1, f2p, p['fp2_w'], p['fp2_b'])
    return _fp3_cls(pts_r, feat_r, oh, p1_t, f1p,
                    p['fp3_w'], p['fp3_b'], p['cls_w'], p['cls_b'])


# speedup vs baseline: 1.0692x; 1.0692x over previous
import functools

import jax
import jax.numpy as jnp
from jax import lax
from jax.experimental import pallas as pl

_F32 = jnp.float32
_DN_T = (((1,), (1,)), ((), ()))
_DN = (((1,), (0,)), ((), ()))


def _dot(x, w):
    return lax.dot_general(x, w, _DN_T, preferred_element_type=_F32)


def _fps_body(npoint, xyz_ref, out_ref):
    xyz = xyz_ref[...]
    b, _, n = xyz.shape
    lane = lax.broadcasted_iota(jnp.int32, (b, 1, n), 2)

    def step(i, state):
        distance, farthest = state
        sel = lane == farthest[:, :, None]
        centroid = jnp.sum(jnp.where(sel, xyz, 0.0), axis=2)
        out_ref[pl.ds(i, 1)] = centroid[None]
        d = xyz - centroid[:, :, None]
        d = d * d
        dist = (d[:, 0, :] + d[:, 1, :]) + d[:, 2, :]
        distance = jnp.minimum(distance, dist)
        farthest = jnp.argmax(distance, axis=1).astype(jnp.int32)[:, None]
        return distance, farthest

    init = (jnp.full((b, n), 1e10, _F32), jnp.zeros((b, 1), jnp.int32))
    lax.fori_loop(0, npoint, step, init)


def _fps(xyz_t, npoint):
    b, _, n = xyz_t.shape
    out = pl.pallas_call(
        functools.partial(_fps_body, npoint),
        out_shape=jax.ShapeDtypeStruct((npoint, b, 3), _F32),
    )(xyz_t)
    return jnp.transpose(out, (1, 0, 2))


def _sa_body(r2, sb, ns, nlayers, *refs):
    pts_ref, xyzr_ref, feat_ref, ctr_ref = refs[:4]
    wrefs = refs[4:4 + 2 * nlayers]
    out_ref = refs[-1]
    pts = pts_ref[0]
    ctr = ctr_ref[0]
    n = pts.shape[1]
    d2 = None
    for c in range(3):
        diff = ctr[:, c:c + 1] - pts[c:c + 1, :]
        t = diff * diff
        d2 = t if d2 is None else d2 + t
    mask = d2 <= r2
    cnt = mask.astype(jnp.int32)
    s = 1
    while s < n:
        cnt = cnt + jnp.concatenate(
            [jnp.zeros((sb, s), jnp.int32), cnt[:, :n - s]], axis=1)
        s *= 2
    rank = jnp.where(mask, cnt - 1, -1)
    counts = cnt[:, n - 1:n]
    kio = lax.broadcasted_iota(jnp.int32, (sb, ns, n), 1)
    oh = (rank[:, None, :] == kio).astype(_F32)
    vals = jnp.concatenate([xyzr_ref[0], feat_ref[0]], axis=1)
    cdim = vals.shape[1]
    g = lax.dot_general(oh.reshape(sb * ns, n), vals, _DN,
                        preferred_element_type=_F32)
    g3 = g.reshape(sb, ns, cdim)
    valid = lax.broadcasted_iota(jnp.int32, (sb, ns, 1), 1) < counts[:, :, None]
    g3 = jnp.where(valid, g3, g3[:, 0:1, :])
    cpad = jnp.concatenate([ctr, jnp.zeros((sb, cdim - 3), _F32)], axis=1)
    h = (g3 - cpad[:, None, :]).reshape(sb * ns, cdim)
    for li in range(nlayers):
        h = jnp.maximum(_dot(h, wrefs[2 * li][...]) + wrefs[2 * li + 1][...], 0.0)
    out_ref[0] = jnp.max(h.reshape(sb, ns, h.shape[1]), axis=1)


def _sa(pts_t, xyz_r, feat_r, ctr, ws, bs, radius, sb):
    b, _, n = pts_t.shape
    s_tot = ctr.shape[1]
    cin = feat_r.shape[2]
    o_last = ws[-1].shape[0]
    in_specs = [
        pl.BlockSpec((1, 3, n), lambda i, j: (i, 0, 0)),
        pl.BlockSpec((1, n, 3), lambda i, j: (i, 0, 0)),
        pl.BlockSpec((1, n, cin), lambda i, j: (i, 0, 0)),
        pl.BlockSpec((1, sb, 3), lambda i, j: (i, j, 0)),
    ]
    args = [pts_t, xyz_r, feat_r, ctr]
    for w, bias in zip(ws, bs):
        args += [w, bias.reshape(1, -1)]
        in_specs += [pl.BlockSpec(w.shape, lambda i, j: (0, 0)),
                     pl.BlockSpec((1, bias.shape[0]), lambda i, j: (0, 0))]
    return pl.pallas_call(
        functools.partial(_sa_body, float(radius) ** 2, sb, 64, len(ws)),
        grid=(b, s_tot // sb),
        in_specs=in_specs,
        out_specs=pl.BlockSpec((1, sb, o_last), lambda i, j: (i, j, 0)),
        out_shape=jax.ShapeDtypeStruct((b, s_tot, o_last), _F32),
    )(*args)


def _gsa_fp1_body(p2_ref, f2_ref, w1p_ref, w1f_ref, b1_ref, w2_ref, b2_ref,
                  w3_ref, b3_ref, a1_ref, ai_ref, c1_ref, a2_ref, c2_ref,
                  out_ref):
    f2 = f2_ref[0]
    h = jnp.maximum(_dot(p2_ref[0], w1p_ref[...])
                    + _dot(f2, w1f_ref[...]) + b1_ref[...], 0.0)
    h = jnp.maximum(_dot(h, w2_ref[...]) + b2_ref[...], 0.0)
    h = jnp.maximum(_dot(h, w3_ref[...]) + b3_ref[...], 0.0)
    f3 = jnp.max(h, axis=0, keepdims=True)
    g = jnp.maximum(_dot(f2, a1_ref[...])
                    + _dot(f3, ai_ref[...]) + c1_ref[...], 0.0)
    g = jnp.maximum(_dot(g, a2_ref[...]) + c2_ref[...], 0.0)
    out_ref[0] = g


def _gsa_fp1(p2_r, f2, gsa_w, gsa_b, fp1_w, fp1_b):
    b = p2_r.shape[0]
    args = [p2_r, f2,
            gsa_w[0][:, :3], gsa_w[0][:, 3:], gsa_b[0].reshape(1, -1),
            gsa_w[1], gsa_b[1].reshape(1, -1),
            gsa_w[2], gsa_b[2].reshape(1, -1),
            fp1_w[0][:, :256], fp1_w[0][:, 256:], fp1_b[0].reshape(1, -1),
            fp1_w[1], fp1_b[1].reshape(1, -1)]
    in_specs = [pl.BlockSpec((1, 128, 3), lambda i: (i, 0, 0)),
                pl.BlockSpec((1, 128, 256), lambda i: (i, 0, 0))]
    in_specs += [pl.BlockSpec(a.shape, lambda i: (0, 0)) for a in args[2:]]
    return pl.pallas_call(
        _gsa_fp1_body,
        grid=(b,),
        in_specs=in_specs,
        out_specs=pl.BlockSpec((1, 128, 256), lambda i: (i, 0, 0)),
        out_shape=jax.ShapeDtypeStruct((b, 128, 256), _F32),
    )(*args)


def _interp3(tgt_r, src_t, src_feat):
    d = None
    for c in range(3):
        diff = tgt_r[:, c:c + 1] - src_t[c:c + 1, :]
        t = diff * diff
        d = t if d is None else d + t
    lio = lax.broadcasted_iota(jnp.int32, d.shape, 1)
    inf = jnp.float32(jnp.inf)
    i1 = jnp.argmin(d, axis=1).astype(jnp.int32)[:, None]
    m1 = jnp.min(d, axis=1)[:, None]
    d2 = jnp.where(lio == i1, inf, d)
    i2 = jnp.argmin(d2, axis=1).astype(jnp.int32)[:, None]
    m2 = jnp.min(d2, axis=1)[:, None]
    d3 = jnp.where(lio == i2, inf, d2)
    i3 = jnp.argmin(d3, axis=1).astype(jnp.int32)[:, None]
    m3 = jnp.min(d3, axis=1)[:, None]
    wa = 1.0 / (m1 + 1e-8)
    wb = 1.0 / (m2 + 1e-8)
    wc = 1.0 / (m3 + 1e-8)
    tot = (wa + wb) + wc
    a = (jnp.where(lio == i1, wa / tot, 0.0)
         + jnp.where(lio == i2, wb / tot, 0.0)
         + jnp.where(lio == i3, wc / tot, 0.0))
    return lax.dot_general(a, src_feat, _DN, preferred_element_type=_F32)


def _fp2_body(p1_ref, p2t_ref, f1_ref, f2p_ref, a1_ref, ai_ref, b1_ref,
              w2_ref, b2_ref, out_ref):
    interp = _interp3(p1_ref[0], p2t_ref[0], f2p_ref[0])
    h = jnp.maximum(_dot(f1_ref[0], a1_ref[...])
                    + _dot(interp, ai_ref[...]) + b1_ref[...], 0.0)
    h = jnp.maximum(_dot(h, w2_ref[...]) + b2_ref[...], 0.0)
    out_ref[0] = h


def _fp2(p1_r, p2_t, f1, f2p, fp2_w, fp2_b):
    b = p1_r.shape[0]
    args = [p1_r, p2_t, f1, f2p,
            fp2_w[0][:, :128], fp2_w[0][:, 128:], fp2_b[0].reshape(1, -1),
            fp2_w[1], fp2_b[1].reshape(1, -1)]
    in_specs = [pl.BlockSpec((1, 512, 3), lambda i: (i, 0, 0)),
                pl.BlockSpec((1, 3, 128), lambda i: (i, 0, 0)),
                pl.BlockSpec((1, 512, 128), lambda i: (i, 0, 0)),
                pl.BlockSpec((1, 128, 256), lambda i: (i, 0, 0))]
    in_specs += [pl.BlockSpec(a.shape, lambda i: (0, 0)) for a in args[4:]]
    return pl.pallas_call(
        _fp2_body,
        grid=(b,),
        in_specs=in_specs,
        out_specs=pl.BlockSpec((1, 512, 128), lambda i: (i, 0, 0)),
        out_shape=jax.ShapeDtypeStruct((b, 512, 128), _F32),
    )(*args)


def _fp3_body(ptsr_ref, featr_ref, oh_ref, p1t_ref, f1p_ref,
              w1o_ref, w1f_ref, w1p_ref, w1i_ref, b1_ref,
              w2_ref, b2_ref, w3_ref, b3_ref,
              c1_ref, cb1_ref, c2_ref, cb2_ref, out_ref):
    ptsr = ptsr_ref[0]
    interp = _interp3(ptsr, p1t_ref[0], f1p_ref[0])
    h = jnp.maximum(_dot(oh_ref[0], w1o_ref[...])
                    + _dot(featr_ref[0], w1f_ref[...])
                    + _dot(ptsr, w1p_ref[...])
                    + _dot(interp, w1i_ref[...]) + b1_ref[...], 0.0)
    h = jnp.maximum(_dot(h, w2_ref[...]) + b2_ref[...], 0.0)
    h = jnp.maximum(_dot(h, w3_ref[...]) + b3_ref[...], 0.0)
    h = jnp.maximum(_dot(h, c1_ref[...]) + cb1_ref[...], 0.0)
    out = lax.dot_general(c2_ref[...], h, _DN_T,
                          preferred_element_type=_F32)
    out_ref[0] = out + cb2_ref[...]


def _fp3_cls(pts_r, feat_r, oh, p1_t, f1p, fp3_w, fp3_b, cls_w, cls_b):
    b, n, _ = pts_r.shape
    w1 = fp3_w[0]
    args = [pts_r, feat_r, oh, p1_t, f1p,
            w1[:, :16], w1[:, 16:19], w1[:, 19:22], w1[:, 22:],
            fp3_b[0].reshape(1, -1),
            fp3_w[1], fp3_b[1].reshape(1, -1),
            fp3_w[2], fp3_b[2].reshape(1, -1),
            cls_w[0], cls_b[0].reshape(1, -1),
            cls_w[1], cls_b[1].reshape(-1, 1)]
    in_specs = [pl.BlockSpec((1, n, 3), lambda i: (i, 0, 0)),
                pl.BlockSpec((1, n, 3), lambda i: (i, 0, 0)),
                pl.BlockSpec((1, 1, 16), lambda i: (i, 0, 0)),
                pl.BlockSpec((1, 3, 512), lambda i: (i, 0, 0)),
                pl.BlockSpec((1, 512, 128), lambda i: (i, 0, 0))]
    in_specs += [pl.BlockSpec(a.shape, lambda i: (0, 0)) for a in args[5:]]
    return pl.pallas_call(
        _fp3_body,
        grid=(b,),
        in_specs=in_specs,
        out_specs=pl.BlockSpec((1, 50, n), lambda i: (i, 0, 0)),
        out_shape=jax.ShapeDtypeStruct((b, 50, n), _F32),
    )(*args)


def kernel(points, features, class_ids, params):
    p = params
    pts_r = jnp.transpose(points, (0, 2, 1))
    feat_r = jnp.transpose(features, (0, 2, 1))
    oh = jax.nn.one_hot(class_ids, 16, dtype=_F32)[:, None, :]

    ctr1 = _fps(points, 512)
    f1 = _sa(points, pts_r, feat_r, ctr1, p['sa1_w'], p['sa1_b'], 0.2, 64)
    p1_t = jnp.transpose(ctr1, (0, 2, 1))
    ctr2 = _fps(p1_t, 128)
    f2 = _sa(p1_t, ctr1, f1, ctr2, p['sa2_w'], p['sa2_b'], 0.4, 64)
    f2p = _gsa_fp1(ctr2, f2, p['gsa_w'], p['gsa_b'], p['fp1_w'], p['fp1_b'])
    p2_t = jnp.transpose(ctr2, (0, 2, 1))
    f1p = _fp2(ctr1, p2_t, f1, f2p, p['fp2_w'], p['fp2_b'])
    return _fp3_cls(pts_r, feat_r, oh, p1_t, f1p,
                    p['fp3_w'], p['fp3_b'], p['cls_w'], p['cls_b'])
